# baseline (device time: 240768 ns/iter reference)
import jax
import jax.numpy as jnp
from jax import lax
from jax.experimental import pallas as pl
from jax.experimental.pallas import tpu as pltpu

N_Y = 4
S = 8
NT = (((1,), (1,)), ((), ()))


def kernel(dy, W):
    m, k = dy.shape
    n = W.shape[0]
    half = m // 2
    sub = half // S
    k2 = k // 2

    def body(dy_ref, w_ref, out_ref, dyv_ref, wv_ref, acc_ref,
             pre_ref, suf_ref, sufout_ref, cop_sems,
             pre_send_sems, pre_recv_sems, suf_send_sems, suf_recv_sems,
             x_send_sems, x_recv_sems):
        my_x = lax.axis_index("x")
        my_y = lax.axis_index("y")
        my_z = lax.axis_index("z")
        other_x = 1 - my_x
        half_off = my_x * half
        other_off = other_x * half
        x_dev = (other_x, my_y, my_z)
        r_dev = (my_x, my_y + 1, my_z)
        l_dev = (my_x, my_y - 1, my_z)

        cw1 = pltpu.make_async_copy(
            w_ref.at[:, pl.ds(0, k2)], wv_ref.at[:, pl.ds(0, k2)],
            cop_sems.at[0])
        cw2 = pltpu.make_async_copy(
            w_ref.at[:, pl.ds(k2, k2)], wv_ref.at[:, pl.ds(k2, k2)],
            cop_sems.at[1])
        cda = pltpu.make_async_copy(
            dy_ref.at[pl.ds(half_off, half // 2), :],
            dyv_ref.at[pl.ds(0, half // 2), :], cop_sems.at[2])
        cdb = pltpu.make_async_copy(
            dy_ref.at[pl.ds(half_off + half // 2, half // 2), :],
            dyv_ref.at[pl.ds(half // 2, half // 2), :], cop_sems.at[3])
        cw1.start()
        cda.start()
        cw2.start()
        cdb.start()

        barrier = pltpu.get_barrier_semaphore()
        for dev in ((my_x, (my_y + 1) % N_Y, my_z),
                    (my_x, (my_y + N_Y - 1) % N_Y, my_z), x_dev):
            pl.semaphore_signal(
                barrier, inc=1,
                device_id=dev, device_id_type=pl.DeviceIdType.MESH,
            )
        pl.semaphore_wait(barrier, 3)

        @pl.when(my_y == 0)
        def _():
            pre_ref[...] = jnp.zeros_like(pre_ref)

        @pl.when(my_y == N_Y - 1)
        def _():
            suf_ref[...] = jnp.zeros_like(suf_ref)

        cw1.wait()
        cda.wait()
        acc_ref[pl.ds(0, half // 2), :] = lax.dot_general(
            dyv_ref[pl.ds(0, half // 2), pl.ds(0, k2)],
            wv_ref[:, pl.ds(0, k2)], NT, preferred_element_type=jnp.float32)
        cw2.wait()
        acc_ref[pl.ds(0, half // 2), :] = acc_ref[pl.ds(0, half // 2), :] + (
            lax.dot_general(
                dyv_ref[pl.ds(0, half // 2), pl.ds(k2, k2)],
                wv_ref[:, pl.ds(k2, k2)], NT,
                preferred_element_type=jnp.float32))
        cdb.wait()
        acc_ref[pl.ds(half // 2, half // 2), :] = lax.dot_general(
            dyv_ref[pl.ds(half // 2, half // 2), pl.ds(0, k2)],
            wv_ref[:, pl.ds(0, k2)], NT, preferred_element_type=jnp.float32)
        acc_ref[pl.ds(half // 2, half // 2), :] = (
            acc_ref[pl.ds(half // 2, half // 2), :]
            + lax.dot_general(
                dyv_ref[pl.ds(half // 2, half // 2), pl.ds(k2, k2)],
                wv_ref[:, pl.ds(k2, k2)], NT,
                preferred_element_type=jnp.float32))

        def pre_rdma(b):
            rows = pl.ds(b * sub, sub)
            return pltpu.make_async_remote_copy(
                src_ref=pre_ref.at[rows, :], dst_ref=pre_ref.at[rows, :],
                send_sem=pre_send_sems.at[b], recv_sem=pre_recv_sems.at[b],
                device_id=r_dev, device_id_type=pl.DeviceIdType.MESH)

        def suf_rdma(b):
            rows = pl.ds(b * sub, sub)
            return pltpu.make_async_remote_copy(
                src_ref=sufout_ref.at[rows, :], dst_ref=suf_ref.at[rows, :],
                send_sem=suf_send_sems.at[b], recv_sem=suf_recv_sems.at[b],
                device_id=l_dev, device_id_type=pl.DeviceIdType.MESH)

        def x_rdma(b, rows):
            return pltpu.make_async_remote_copy(
                src_ref=out_ref.at[rows, :], dst_ref=out_ref.at[rows, :],
                send_sem=x_send_sems.at[b], recv_sem=x_recv_sems.at[b],
                device_id=x_dev, device_id_type=pl.DeviceIdType.MESH)

        for b in range(S):
            rows = pl.ds(b * sub, sub)

            @pl.when(my_y > 0)
            def _(b=b):
                pre_rdma(b).wait_recv()
            pre_ref[rows, :] = pre_ref[rows, :] + acc_ref[rows, :]

            @pl.when(my_y < N_Y - 1)
            def _(b=b):
                pre_rdma(b).start()

            @pl.when(my_y < N_Y - 1)
            def _(b=b):
                suf_rdma(b).wait_recv()
            sufout_ref[rows, :] = suf_ref[rows, :] + acc_ref[rows, :]

            @pl.when(my_y > 0)
            def _(b=b):
                suf_rdma(b).start()

            orows = pl.ds(half_off + b * sub, sub)
            out_ref[orows, :] = pre_ref[rows, :] + suf_ref[rows, :]
            x_rdma(b, orows).start()

        for b in range(S):
            x_rdma(b, pl.ds(other_off + b * sub, sub)).wait_recv()
        for b in range(S):
            x_rdma(b, pl.ds(half_off + b * sub, sub)).wait_send()

            @pl.when(my_y < N_Y - 1)
            def _(b=b):
                pre_rdma(b).wait_send()

            @pl.when(my_y > 0)
            def _(b=b):
                suf_rdma(b).wait_send()

    return pl.pallas_call(
        body,
        out_shape=jax.ShapeDtypeStruct((m, n), jnp.float32),
        in_specs=[
            pl.BlockSpec(memory_space=pl.ANY),
            pl.BlockSpec(memory_space=pl.ANY),
        ],
        out_specs=pl.BlockSpec(memory_space=pltpu.VMEM),
        scratch_shapes=[
            pltpu.VMEM((half, k), jnp.float32),
            pltpu.VMEM((n, k), jnp.float32),
            pltpu.VMEM((half, n), jnp.float32),
            pltpu.VMEM((half, n), jnp.float32),
            pltpu.VMEM((half, n), jnp.float32),
            pltpu.VMEM((half, n), jnp.float32),
            pltpu.SemaphoreType.DMA((4,)),
            pltpu.SemaphoreType.DMA((S,)),
            pltpu.SemaphoreType.DMA((S,)),
            pltpu.SemaphoreType.DMA((S,)),
            pltpu.SemaphoreType.DMA((S,)),
            pltpu.SemaphoreType.DMA((S,)),
            pltpu.SemaphoreType.DMA((S,)),
        ],
        compiler_params=pltpu.CompilerParams(collective_id=0),
    )(dy, W)


# device time: 63674 ns/iter; 3.7813x vs baseline; 3.7813x over previous
import jax
import jax.numpy as jnp
from jax import lax
from jax.experimental import pallas as pl
from jax.experimental.pallas import tpu as pltpu

N_Y = 4
SUB = 2
NT = (((1,), (1,)), ((), ()))


def kernel(dy, W):
    m, k = dy.shape
    n = W.shape[0]
    half = m // 2
    ch = half // N_Y
    sch = ch // SUB
    k2 = k // 2

    def body(dy_ref, w_ref, out_ref, dyv_ref, wv_ref, acc_ref,
             send_ref, rs_recv_ref, cop_sems,
             rs_send_sems, rs_recv_sems, ag_send_sems, ag_recv_sems,
             x_send_sems, x_recv_sems):
        my_x = lax.axis_index("x")
        my_y = lax.axis_index("y")
        my_z = lax.axis_index("z")
        right = (my_y + 1) % N_Y
        left = (my_y + N_Y - 1) % N_Y
        other_x = 1 - my_x
        half_off = my_x * half
        other_off = other_x * half
        y_dev = (my_x, right, my_z)
        x_dev = (other_x, my_y, my_z)

        def chunk_c(j):
            return (my_y + N_Y - j) % N_Y

        cw1 = pltpu.make_async_copy(
            w_ref.at[:, pl.ds(0, k2)], wv_ref.at[:, pl.ds(0, k2)],
            cop_sems.at[0])
        cw2 = pltpu.make_async_copy(
            w_ref.at[:, pl.ds(k2, k2)], wv_ref.at[:, pl.ds(k2, k2)],
            cop_sems.at[1])
        cw1.start()
        cds = []
        for j in range(N_Y):
            c = pltpu.make_async_copy(
                dy_ref.at[pl.ds(half_off + chunk_c(j) * ch, ch), :],
                dyv_ref.at[pl.ds(j * ch, ch), :], cop_sems.at[2 + j])
            c.start()
            cds.append(c)
        cw2.start()

        barrier = pltpu.get_barrier_semaphore()
        for dev in ((my_x, left, my_z), y_dev, x_dev):
            pl.semaphore_signal(
                barrier, inc=1,
                device_id=dev, device_id_type=pl.DeviceIdType.MESH,
            )
        pl.semaphore_wait(barrier, 3)

        cw1.wait()
        cds[0].wait()
        cds[1].wait()
        acc_ref[pl.ds(0, 2 * ch), :] = lax.dot_general(
            dyv_ref[pl.ds(0, 2 * ch), pl.ds(0, k2)], wv_ref[:, pl.ds(0, k2)],
            NT, preferred_element_type=jnp.float32)
        cw2.wait()
        acc_ref[pl.ds(0, 2 * ch), :] = acc_ref[pl.ds(0, 2 * ch), :] + (
            lax.dot_general(
                dyv_ref[pl.ds(0, 2 * ch), pl.ds(k2, k2)],
                wv_ref[:, pl.ds(k2, k2)], NT,
                preferred_element_type=jnp.float32))

        def rs_rdma(s, b):
            return pltpu.make_async_remote_copy(
                src_ref=send_ref.at[s, b],
                dst_ref=rs_recv_ref.at[s, b],
                send_sem=rs_send_sems.at[s, b],
                recv_sem=rs_recv_sems.at[s, b],
                device_id=y_dev, device_id_type=pl.DeviceIdType.MESH,
            )

        def ag_rdma(s, b, rows):
            return pltpu.make_async_remote_copy(
                src_ref=out_ref.at[rows, :],
                dst_ref=out_ref.at[rows, :],
                send_sem=ag_send_sems.at[s, b],
                recv_sem=ag_recv_sems.at[s, b],
                device_id=y_dev, device_id_type=pl.DeviceIdType.MESH,
            )

        def x_rdma(j, rows):
            return pltpu.make_async_remote_copy(
                src_ref=out_ref.at[rows, :],
                dst_ref=out_ref.at[rows, :],
                send_sem=x_send_sems.at[j],
                recv_sem=x_recv_sems.at[j],
                device_id=x_dev, device_id_type=pl.DeviceIdType.MESH,
            )

        rs = [[None] * SUB for _ in range(N_Y - 1)]
        for b in range(SUB):
            send_ref[0, b] = acc_ref[pl.ds(b * sch, sch), :]
            rs[0][b] = rs_rdma(0, b)
            rs[0][b].start()

        cds[2].wait()
        cds[3].wait()
        acc_ref[pl.ds(2 * ch, 2 * ch), :] = lax.dot_general(
            dyv_ref[pl.ds(2 * ch, 2 * ch), pl.ds(0, k2)],
            wv_ref[:, pl.ds(0, k2)], NT, preferred_element_type=jnp.float32)
        acc_ref[pl.ds(2 * ch, 2 * ch), :] = (
            acc_ref[pl.ds(2 * ch, 2 * ch), :]
            + lax.dot_general(
                dyv_ref[pl.ds(2 * ch, 2 * ch), pl.ds(k2, k2)],
                wv_ref[:, pl.ds(k2, k2)], NT,
                preferred_element_type=jnp.float32))

        for s in range(1, N_Y - 1):
            for b in range(SUB):
                rs[s - 1][b].wait_recv()
                send_ref[s, b] = (
                    acc_ref[pl.ds(s * ch + b * sch, sch), :]
                    + rs_recv_ref[s - 1, b]
                )
                rs[s][b] = rs_rdma(s, b)
                rs[s][b].start()

        c_own = chunk_c(3)
        for b in range(SUB):
            rs[N_Y - 2][b].wait_recv()
            out_ref[pl.ds(half_off + c_own * ch + b * sch, sch), :] = (
                acc_ref[pl.ds(3 * ch + b * sch, sch), :]
                + rs_recv_ref[N_Y - 2, b]
            )

        x_sends = []
        x0 = x_rdma(0, pl.ds(half_off + c_own * ch, ch))
        x0.start()
        x_sends.append(x0)

        ag = [[None] * SUB for _ in range(N_Y - 1)]
        for b in range(SUB):
            ag[0][b] = ag_rdma(0, b, pl.ds(half_off + c_own * ch + b * sch, sch))
            ag[0][b].start()
        for s in range(1, N_Y - 1):
            c = (my_y + 1 + N_Y - s) % N_Y
            for b in range(SUB):
                ag[s - 1][b].wait_recv()
                ag[s][b] = ag_rdma(s, b, pl.ds(half_off + c * ch + b * sch, sch))
                ag[s][b].start()
            c_fin = (my_y + N_Y - (s - 1)) % N_Y
            xs = x_rdma(s, pl.ds(half_off + c_fin * ch, ch))
            xs.start()
            x_sends.append(xs)
        for b in range(SUB):
            ag[N_Y - 2][b].wait_recv()
        c_fin = (my_y + N_Y - (N_Y - 2)) % N_Y
        xl = x_rdma(N_Y - 1, pl.ds(half_off + c_fin * ch, ch))
        xl.start()
        x_sends.append(xl)

        for j in range(N_Y):
            c_j = c_own if j == 0 else (my_y + N_Y - (j - 1)) % N_Y
            x_rdma(j, pl.ds(other_off + c_j * ch, ch)).wait_recv()
        for d in x_sends:
            d.wait_send()
        for s in range(N_Y - 1):
            for b in range(SUB):
                rs[s][b].wait_send()
                ag[s][b].wait_send()

    return pl.pallas_call(
        body,
        out_shape=jax.ShapeDtypeStruct((m, n), jnp.float32),
        in_specs=[
            pl.BlockSpec(memory_space=pl.ANY),
            pl.BlockSpec(memory_space=pl.ANY),
        ],
        out_specs=pl.BlockSpec(memory_space=pltpu.VMEM),
        scratch_shapes=[
            pltpu.VMEM((half, k), jnp.float32),
            pltpu.VMEM((n, k), jnp.float32),
            pltpu.VMEM((half, n), jnp.float32),
            pltpu.VMEM((N_Y - 1, SUB, sch, n), jnp.float32),
            pltpu.VMEM((N_Y - 1, SUB, sch, n), jnp.float32),
            pltpu.SemaphoreType.DMA((6,)),
            pltpu.SemaphoreType.DMA((N_Y - 1, SUB)),
            pltpu.SemaphoreType.DMA((N_Y - 1, SUB)),
            pltpu.SemaphoreType.DMA((N_Y - 1, SUB)),
            pltpu.SemaphoreType.DMA((N_Y - 1, SUB)),
            pltpu.SemaphoreType.DMA((N_Y,)),
            pltpu.SemaphoreType.DMA((N_Y,)),
        ],
        compiler_params=pltpu.CompilerParams(collective_id=0),
    )(dy, W)


# device time: 59402 ns/iter; 4.0532x vs baseline; 1.0719x over previous
import jax
import jax.numpy as jnp
from jax import lax
from jax.experimental import pallas as pl
from jax.experimental.pallas import tpu as pltpu

N_Y = 4
S = 8
LAG = 2
NT = (((1,), (1,)), ((), ()))


def kernel(dy, W):
    m, k = dy.shape
    n = W.shape[0]
    half = m // 2
    sub = half // S
    k2 = k // 2

    def body(dy_ref, w_ref, out_ref, dyv_ref, wv_ref, acc_ref,
             pre_ref, suf_ref, sufout_ref, cop_sems,
             pre_send_sems, pre_recv_sems, suf_send_sems, suf_recv_sems,
             x_send_sems, x_recv_sems):
        my_x = lax.axis_index("x")
        my_y = lax.axis_index("y")
        my_z = lax.axis_index("z")
        other_x = 1 - my_x
        half_off = my_x * half
        other_off = other_x * half
        x_dev = (other_x, my_y, my_z)
        r_dev = (my_x, my_y + 1, my_z)
        l_dev = (my_x, my_y - 1, my_z)

        cw1 = pltpu.make_async_copy(
            w_ref.at[:, pl.ds(0, k2)], wv_ref.at[:, pl.ds(0, k2)],
            cop_sems.at[0])
        cw2 = pltpu.make_async_copy(
            w_ref.at[:, pl.ds(k2, k2)], wv_ref.at[:, pl.ds(k2, k2)],
            cop_sems.at[1])
        cda = pltpu.make_async_copy(
            dy_ref.at[pl.ds(half_off, half // 2), :],
            dyv_ref.at[pl.ds(0, half // 2), :], cop_sems.at[2])
        cdb = pltpu.make_async_copy(
            dy_ref.at[pl.ds(half_off + half // 2, half // 2), :],
            dyv_ref.at[pl.ds(half // 2, half // 2), :], cop_sems.at[3])
        cw1.start()
        cda.start()
        cw2.start()
        cdb.start()

        barrier = pltpu.get_barrier_semaphore()
        for dev in ((my_x, (my_y + 1) % N_Y, my_z),
                    (my_x, (my_y + N_Y - 1) % N_Y, my_z), x_dev):
            pl.semaphore_signal(
                barrier, inc=1,
                device_id=dev, device_id_type=pl.DeviceIdType.MESH,
            )
        pl.semaphore_wait(barrier, 3)

        cw1.wait()
        cda.wait()
        acc_ref[pl.ds(0, half // 2), :] = lax.dot_general(
            dyv_ref[pl.ds(0, half // 2), pl.ds(0, k2)],
            wv_ref[:, pl.ds(0, k2)], NT, preferred_element_type=jnp.float32)
        cw2.wait()
        acc_ref[pl.ds(0, half // 2), :] = acc_ref[pl.ds(0, half // 2), :] + (
            lax.dot_general(
                dyv_ref[pl.ds(0, half // 2), pl.ds(k2, k2)],
                wv_ref[:, pl.ds(k2, k2)], NT,
                preferred_element_type=jnp.float32))
        cdb.wait()
        acc_ref[pl.ds(half // 2, half // 2), :] = lax.dot_general(
            dyv_ref[pl.ds(half // 2, half // 2), pl.ds(0, k2)],
            wv_ref[:, pl.ds(0, k2)], NT, preferred_element_type=jnp.float32)
        acc_ref[pl.ds(half // 2, half // 2), :] = (
            acc_ref[pl.ds(half // 2, half // 2), :]
            + lax.dot_general(
                dyv_ref[pl.ds(half // 2, half // 2), pl.ds(k2, k2)],
                wv_ref[:, pl.ds(k2, k2)], NT,
                preferred_element_type=jnp.float32))

        def rows(b):
            return pl.ds(b * sub, sub)

        def pre_rdma(b):
            return pltpu.make_async_remote_copy(
                src_ref=pre_ref.at[rows(b), :], dst_ref=pre_ref.at[rows(b), :],
                send_sem=pre_send_sems.at[b], recv_sem=pre_recv_sems.at[b],
                device_id=r_dev, device_id_type=pl.DeviceIdType.MESH)

        def suf_rdma(b):
            return pltpu.make_async_remote_copy(
                src_ref=sufout_ref.at[rows(b), :],
                dst_ref=suf_ref.at[rows(b), :],
                send_sem=suf_send_sems.at[b], recv_sem=suf_recv_sems.at[b],
                device_id=l_dev, device_id_type=pl.DeviceIdType.MESH)

        def x_rdma(b, r):
            return pltpu.make_async_remote_copy(
                src_ref=out_ref.at[r, :], dst_ref=out_ref.at[r, :],
                send_sem=x_send_sems.at[b], recv_sem=x_recv_sems.at[b],
                device_id=x_dev, device_id_type=pl.DeviceIdType.MESH)

        def finish(b):
            orow = pl.ds(half_off + b * sub, sub)
            out_ref[orow, :] = pre_ref[rows(b), :] + suf_ref[rows(b), :]
            x_rdma(b, orow).start()

        @pl.when(my_y == 0)
        def _():
            for b in range(S):
                pre_ref[rows(b), :] = acc_ref[rows(b), :]
                pre_rdma(b).start()
            for b in range(S):
                suf_rdma(b).wait_recv()
                finish(b)

        @pl.when(my_y == 1)
        def _():
            for t in range(S + LAG):
                if t < S:
                    pre_rdma(t).wait_recv()
                    pre_ref[rows(t), :] = pre_ref[rows(t), :] + acc_ref[rows(t), :]
                    pre_rdma(t).start()
                b = t - LAG
                if 0 <= b < S:
                    suf_rdma(b).wait_recv()
                    sufout_ref[rows(b), :] = suf_ref[rows(b), :] + acc_ref[rows(b), :]
                    suf_rdma(b).start()
                    finish(b)

        @pl.when(my_y == 2)
        def _():
            for t in range(S + LAG):
                if t < S:
                    suf_rdma(t).wait_recv()
                    sufout_ref[rows(t), :] = suf_ref[rows(t), :] + acc_ref[rows(t), :]
                    suf_rdma(t).start()
                b = t - LAG
                if 0 <= b < S:
                    pre_rdma(b).wait_recv()
                    pre_ref[rows(b), :] = pre_ref[rows(b), :] + acc_ref[rows(b), :]
                    pre_rdma(b).start()
                    finish(b)

        @pl.when(my_y == N_Y - 1)
        def _():
            for b in range(S):
                sufout_ref[rows(b), :] = acc_ref[rows(b), :]
                suf_rdma(b).start()
            for b in range(S):
                pre_rdma(b).wait_recv()
                pre_ref[rows(b), :] = pre_ref[rows(b), :] + acc_ref[rows(b), :]
                orow = pl.ds(half_off + b * sub, sub)
                out_ref[orow, :] = pre_ref[rows(b), :]
                x_rdma(b, orow).start()

        for b in range(S):
            x_rdma(b, pl.ds(other_off + b * sub, sub)).wait_recv()
        for b in range(S):
            x_rdma(b, pl.ds(half_off + b * sub, sub)).wait_send()

            @pl.when(my_y < N_Y - 1)
            def _(b=b):
                pre_rdma(b).wait_send()

            @pl.when(my_y > 0)
            def _(b=b):
                suf_rdma(b).wait_send()

    return pl.pallas_call(
        body,
        out_shape=jax.ShapeDtypeStruct((m, n), jnp.float32),
        in_specs=[
            pl.BlockSpec(memory_space=pl.ANY),
            pl.BlockSpec(memory_space=pl.ANY),
        ],
        out_specs=pl.BlockSpec(memory_space=pltpu.VMEM),
        scratch_shapes=[
            pltpu.VMEM((half, k), jnp.float32),
            pltpu.VMEM((n, k), jnp.float32),
            pltpu.VMEM((half, n), jnp.float32),
            pltpu.VMEM((half, n), jnp.float32),
            pltpu.VMEM((half, n), jnp.float32),
            pltpu.VMEM((half, n), jnp.float32),
            pltpu.SemaphoreType.DMA((4,)),
            pltpu.SemaphoreType.DMA((S,)),
            pltpu.SemaphoreType.DMA((S,)),
            pltpu.SemaphoreType.DMA((S,)),
            pltpu.SemaphoreType.DMA((S,)),
            pltpu.SemaphoreType.DMA((S,)),
            pltpu.SemaphoreType.DMA((S,)),
        ],
        compiler_params=pltpu.CompilerParams(collective_id=0),
    )(dy, W)


# device time: 53829 ns/iter; 4.4728x vs baseline; 1.1035x over previous
import jax
import jax.numpy as jnp
from jax import lax
from jax.experimental import pallas as pl
from jax.experimental.pallas import tpu as pltpu

N_Y = 4
S = 32
LAG = 5
NT = (((1,), (1,)), ((), ()))


def kernel(dy, W):
    m, k = dy.shape
    n = W.shape[0]
    half = m // 2
    sub = half // S
    k2 = k // 2

    def body(dy_ref, w_ref, out_ref, dyv_ref, wv_ref, acc_ref,
             pre_ref, suf_ref, sufout_ref, cop_sems,
             pre_send_sems, pre_recv_sems, suf_send_sems, suf_recv_sems,
             x_send_sems, x_recv_sems):
        my_x = lax.axis_index("x")
        my_y = lax.axis_index("y")
        my_z = lax.axis_index("z")
        other_x = 1 - my_x
        half_off = my_x * half
        other_off = other_x * half
        x_dev = (other_x, my_y, my_z)
        r_dev = (my_x, my_y + 1, my_z)
        l_dev = (my_x, my_y - 1, my_z)

        cw1 = pltpu.make_async_copy(
            w_ref.at[:, pl.ds(0, k2)], wv_ref.at[:, pl.ds(0, k2)],
            cop_sems.at[0])
        cw2 = pltpu.make_async_copy(
            w_ref.at[:, pl.ds(k2, k2)], wv_ref.at[:, pl.ds(k2, k2)],
            cop_sems.at[1])
        cda = pltpu.make_async_copy(
            dy_ref.at[pl.ds(half_off, half // 2), :],
            dyv_ref.at[pl.ds(0, half // 2), :], cop_sems.at[2])
        cdb = pltpu.make_async_copy(
            dy_ref.at[pl.ds(half_off + half // 2, half // 2), :],
            dyv_ref.at[pl.ds(half // 2, half // 2), :], cop_sems.at[3])
        cw1.start()
        cda.start()
        cw2.start()
        cdb.start()

        barrier = pltpu.get_barrier_semaphore()
        for dev in ((my_x, (my_y + 1) % N_Y, my_z),
                    (my_x, (my_y + N_Y - 1) % N_Y, my_z), x_dev):
            pl.semaphore_signal(
                barrier, inc=1,
                device_id=dev, device_id_type=pl.DeviceIdType.MESH,
            )
        pl.semaphore_wait(barrier, 3)

        cw1.wait()
        cda.wait()
        acc_ref[pl.ds(0, half // 2), :] = lax.dot_general(
            dyv_ref[pl.ds(0, half // 2), pl.ds(0, k2)],
            wv_ref[:, pl.ds(0, k2)], NT, preferred_element_type=jnp.float32)
        cw2.wait()
        acc_ref[pl.ds(0, half // 2), :] = acc_ref[pl.ds(0, half // 2), :] + (
            lax.dot_general(
                dyv_ref[pl.ds(0, half // 2), pl.ds(k2, k2)],
                wv_ref[:, pl.ds(k2, k2)], NT,
                preferred_element_type=jnp.float32))

        def dot_b():
            cdb.wait()
            acc_ref[pl.ds(half // 2, half // 2), :] = lax.dot_general(
                dyv_ref[pl.ds(half // 2, half // 2), pl.ds(0, k2)],
                wv_ref[:, pl.ds(0, k2)], NT,
                preferred_element_type=jnp.float32)
            acc_ref[pl.ds(half // 2, half // 2), :] = (
                acc_ref[pl.ds(half // 2, half // 2), :]
                + lax.dot_general(
                    dyv_ref[pl.ds(half // 2, half // 2), pl.ds(k2, k2)],
                    wv_ref[:, pl.ds(k2, k2)], NT,
                    preferred_element_type=jnp.float32))

        def rows(b):
            return pl.ds(b * sub, sub)

        def pre_rdma(b):
            return pltpu.make_async_remote_copy(
                src_ref=pre_ref.at[rows(b), :], dst_ref=pre_ref.at[rows(b), :],
                send_sem=pre_send_sems.at[b], recv_sem=pre_recv_sems.at[b],
                device_id=r_dev, device_id_type=pl.DeviceIdType.MESH)

        def suf_rdma(b):
            return pltpu.make_async_remote_copy(
                src_ref=sufout_ref.at[rows(b), :],
                dst_ref=suf_ref.at[rows(b), :],
                send_sem=suf_send_sems.at[b], recv_sem=suf_recv_sems.at[b],
                device_id=l_dev, device_id_type=pl.DeviceIdType.MESH)

        def x_rdma(b, r):
            return pltpu.make_async_remote_copy(
                src_ref=out_ref.at[r, :], dst_ref=out_ref.at[r, :],
                send_sem=x_send_sems.at[b], recv_sem=x_recv_sems.at[b],
                device_id=x_dev, device_id_type=pl.DeviceIdType.MESH)

        def finish(b):
            orow = pl.ds(half_off + b * sub, sub)
            out_ref[orow, :] = pre_ref[rows(b), :] + suf_ref[rows(b), :]
            x_rdma(b, orow).start()

        def y0_part(lo, hi):
            @pl.when(my_y == 0)
            def _():
                for b in range(lo, hi):
                    pre_ref[rows(b), :] = acc_ref[rows(b), :]
                    pre_rdma(b).start()

        def y1_part(lo, hi):
            @pl.when(my_y == 1)
            def _():
                for t in range(lo, hi):
                    if t < S:
                        pre_rdma(t).wait_recv()
                        pre_ref[rows(t), :] = (
                            pre_ref[rows(t), :] + acc_ref[rows(t), :])
                        pre_rdma(t).start()
                    b = t - LAG
                    if 0 <= b < S:
                        suf_rdma(b).wait_recv()
                        sufout_ref[rows(b), :] = (
                            suf_ref[rows(b), :] + acc_ref[rows(b), :])
                        suf_rdma(b).start()
                        finish(b)

        def y2_part(lo, hi):
            @pl.when(my_y == 2)
            def _():
                for t in range(lo, hi):
                    if t < S:
                        suf_rdma(t).wait_recv()
                        sufout_ref[rows(t), :] = (
                            suf_ref[rows(t), :] + acc_ref[rows(t), :])
                        suf_rdma(t).start()
                    b = t - LAG
                    if 0 <= b < S:
                        pre_rdma(b).wait_recv()
                        pre_ref[rows(b), :] = (
                            pre_ref[rows(b), :] + acc_ref[rows(b), :])
                        pre_rdma(b).start()
                        finish(b)

        def y3_part(lo, hi):
            @pl.when(my_y == N_Y - 1)
            def _():
                for b in range(lo, hi):
                    sufout_ref[rows(b), :] = acc_ref[rows(b), :]
                    suf_rdma(b).start()

        y0_part(0, S // 2)
        y1_part(0, S // 2)
        y2_part(0, S // 2)
        y3_part(0, S // 2)

        dot_b()

        y0_part(S // 2, S)
        y1_part(S // 2, S + LAG)
        y2_part(S // 2, S + LAG)
        y3_part(S // 2, S)

        @pl.when(my_y == 0)
        def _():
            for b in range(S):
                suf_rdma(b).wait_recv()
                finish(b)

        @pl.when(my_y == N_Y - 1)
        def _():
            for b in range(S):
                pre_rdma(b).wait_recv()
                pre_ref[rows(b), :] = pre_ref[rows(b), :] + acc_ref[rows(b), :]
                orow = pl.ds(half_off + b * sub, sub)
                out_ref[orow, :] = pre_ref[rows(b), :]
                x_rdma(b, orow).start()

        for b in range(S):
            x_rdma(b, pl.ds(other_off + b * sub, sub)).wait_recv()
        for b in range(S):
            x_rdma(b, pl.ds(half_off + b * sub, sub)).wait_send()

            @pl.when(my_y < N_Y - 1)
            def _(b=b):
                pre_rdma(b).wait_send()

            @pl.when(my_y > 0)
            def _(b=b):
                suf_rdma(b).wait_send()

    return pl.pallas_call(
        body,
        out_shape=jax.ShapeDtypeStruct((m, n), jnp.float32),
        in_specs=[
            pl.BlockSpec(memory_space=pl.ANY),
            pl.BlockSpec(memory_space=pl.ANY),
        ],
        out_specs=pl.BlockSpec(memory_space=pltpu.VMEM),
        scratch_shapes=[
            pltpu.VMEM((half, k), jnp.float32),
            pltpu.VMEM((n, k), jnp.float32),
            pltpu.VMEM((half, n), jnp.float32),
            pltpu.VMEM((half, n), jnp.float32),
            pltpu.VMEM((half, n), jnp.float32),
            pltpu.VMEM((half, n), jnp.float32),
            pltpu.SemaphoreType.DMA((4,)),
            pltpu.SemaphoreType.DMA((S,)),
            pltpu.SemaphoreType.DMA((S,)),
            pltpu.SemaphoreType.DMA((S,)),
            pltpu.SemaphoreType.DMA((S,)),
            pltpu.SemaphoreType.DMA((S,)),
            pltpu.SemaphoreType.DMA((S,)),
        ],
        compiler_params=pltpu.CompilerParams(collective_id=0),
    )(dy, W)
